# Initial kernel scaffold; baseline (speedup 1.0000x reference)
import jax
import jax.numpy as jnp
from jax.experimental import pallas as pl


def kernel(x, emb, W1, b1, W2, b2):
    raise NotImplementedError("write your pallas kernel here")


# speedup vs baseline: 3.5027x; 3.5027x over previous
import functools

import jax
import jax.numpy as jnp
from jax import lax
from jax.experimental import pallas as pl
from jax.experimental.pallas import tpu as pltpu
from jax.experimental.pallas import tpu_sc as plsc

NC, NS = 2, 16
NW = NC * NS
CHUNK = 512


def _sc_gather(table, idx):
    N = idx.shape[0]
    E = table.shape[1]
    per_w = N // NW
    n_chunks = per_w // CHUNK
    mesh = plsc.VectorSubcoreMesh(core_axis_name="c", subcore_axis_name="s")

    @functools.partial(
        pl.kernel,
        out_type=jax.ShapeDtypeStruct((N, E), table.dtype),
        mesh=mesh,
        scratch_types=[
            pltpu.VMEM((CHUNK,), jnp.int32),
            pltpu.VMEM((CHUNK, E), table.dtype),
            pltpu.SemaphoreType.DMA,
        ],
    )
    def gather_kernel(table_hbm, idx_hbm, out_hbm, idx_v, rows_v, gsem):
        wid = lax.axis_index("s") * NC + lax.axis_index("c")
        base = pl.multiple_of(wid * per_w, CHUNK)

        def body(c, carry):
            off = pl.multiple_of(base + c * CHUNK, CHUNK)
            pltpu.sync_copy(idx_hbm.at[pl.ds(off, CHUNK)], idx_v)
            pltpu.async_copy(table_hbm.at[idx_v], rows_v, gsem).wait()
            pltpu.sync_copy(rows_v, out_hbm.at[pl.ds(off, CHUNK)])
            return carry

        lax.fori_loop(0, n_chunks, body, 0)

    return gather_kernel(table, idx)


def _tc_mlp(h, W1, b1, W2, b2):
    B, K = h.shape
    H = W1.shape[0]
    BM = 128
    grid = (B // BM,)

    def body(hb, w1, b1r, w2, b2r, ob):
        acc = lax.dot_general(hb[...], w1[...], (((1,), (1,)), ((), ())),
                              preferred_element_type=jnp.float32)
        hrelu = jnp.maximum(acc + b1r[...], 0.0)
        s = jnp.sum(hrelu * w2[...], axis=1, keepdims=True)
        ob[...] = s + b2r[0, 0]

    return pl.pallas_call(
        body,
        grid=grid,
        in_specs=[
            pl.BlockSpec((BM, K), lambda i: (i, 0)),
            pl.BlockSpec((H, K), lambda i: (0, 0)),
            pl.BlockSpec((1, H), lambda i: (0, 0)),
            pl.BlockSpec((1, H), lambda i: (0, 0)),
            pl.BlockSpec((1, 1), lambda i: (0, 0)),
        ],
        out_specs=pl.BlockSpec((BM, 1), lambda i: (i, 0)),
        out_shape=jax.ShapeDtypeStruct((B, 1), jnp.float32),
    )(h, W1, b1.reshape(1, H), W2, b2.reshape(1, 1))


def kernel(x, emb, W1, b1, W2, b2):
    B, S = x.shape
    V, E = emb.shape
    idx = x.reshape(-1).astype(jnp.int32)
    g = _sc_gather(emb, idx)
    h = g.reshape(B, S * E)
    return _tc_mlp(h, W1, b1, W2, b2)
